# baseline (speedup 1.0000x reference)
import functools

import jax
import jax.numpy as jnp
from jax.experimental import pallas as pl

N_NODES = 10000
N_EDGES = 160000
N_LG_EDGES = 640000
NODE_F = 64
EDGE_F = 40
ANGLE_F = 40
HIDDEN = 64

_LOG2 = 0.6931471805599453


def _sp(x):
    return jnp.logaddexp(x, 0.0)


def _edge_mlp_body(z_ref, w1_ref, b1_ref, w2_ref, b2_ref, o_ref, *, shifted):
    z = z_ref[...]
    h = jnp.dot(z, w1_ref[...], preferred_element_type=jnp.float32) + b1_ref[...]
    h = _sp(h)
    if shifted:
        h = h - _LOG2
    h = jnp.dot(h, w2_ref[...], preferred_element_type=jnp.float32) + b2_ref[...]
    if shifted:
        h = _sp(h) - _LOG2
    o_ref[...] = h


def _edge_mlp(z, p1, p2, shifted, block=8000):
    r, f = z.shape
    w1, b1 = p1
    w2, b2 = p2
    hid = w1.shape[0]
    out = w2.shape[0]
    return pl.pallas_call(
        functools.partial(_edge_mlp_body, shifted=shifted),
        grid=(r // block,),
        in_specs=[
            pl.BlockSpec((block, f), lambda i: (i, 0)),
            pl.BlockSpec((f, hid), lambda i: (0, 0)),
            pl.BlockSpec((1, hid), lambda i: (0, 0)),
            pl.BlockSpec((hid, out), lambda i: (0, 0)),
            pl.BlockSpec((1, out), lambda i: (0, 0)),
        ],
        out_specs=pl.BlockSpec((block, out), lambda i: (i, 0)),
        out_shape=jax.ShapeDtypeStruct((r, out), jnp.float32),
    )(z, w1.T, b1[None], w2.T, b2[None])


def _rbf(d, vmin, vmax, bins):
    centers = jnp.linspace(vmin, vmax, bins)
    gamma = 1.0 / ((vmax - vmin) / (bins - 1))
    return jnp.exp(-gamma * (d[:, None] - centers) ** 2)


def _linear(x, wb):
    w, b = wb
    return x @ w.T + b


def _batchnorm(x, gamma, beta, eps=1e-5):
    mu = x.mean(axis=0)
    var = x.var(axis=0)
    return gamma * (x - mu) / jnp.sqrt(var + eps) + beta


def kernel(atom_features, r, h_angle, edge_index, lg_edge_index, params):
    src, dst = edge_index[0], edge_index[1]
    lsrc, ldst = lg_edge_index[0], lg_edge_index[1]
    bondlength = jnp.sqrt(jnp.sum(r * r, axis=1))
    edge_feats = _rbf(bondlength, 0.0, 8.0, EDGE_F)
    x = _linear(atom_features, params['atom_emb'])
    x = jax.nn.relu(_batchnorm(x, params['bn'][0], params['bn'][1]))
    z = _rbf(h_angle, -1.0, 1.0, ANGLE_F)

    zp1 = _edge_mlp(z, params['conv1']['lg']['pe1'], params['conv1']['lg']['pe2'], True)
    zp2 = _edge_mlp(z, params['conv2']['lg']['pe1'], params['conv2']['lg']['pe2'], True)

    def clgn(p, x_in, y, zp):
        hv = _linear(x_in, p['pn'])
        he = _edge_mlp(y, p['pe1'], p['pe2'], False)
        m = hv[src] * he
        agg = jax.ops.segment_sum(m, dst, num_segments=N_NODES)
        x_out = _sp(_linear(agg, p['po']))
        y_cat = jnp.concatenate([y, m], axis=1)
        lgp = p['lg']
        hv2 = _linear(y_cat, lgp['pn'])
        m2 = hv2[lsrc] * zp
        h2 = jax.ops.segment_sum(m2, ldst, num_segments=N_EDGES)
        y_out = _sp(_sp(_linear(h2, lgp['po'])) - _LOG2)
        return x_out, y_out

    x1, y1 = clgn(params['conv1'], x, edge_feats, zp1)
    x2, y2 = clgn(params['conv2'], x1, y1, zp2)

    p3 = params['conv3']
    hv = _linear(x2, p3['pn'])
    he = _edge_mlp(y2, p3['pe1'], p3['pe2'], True)
    m = hv[src] * he
    h = jax.ops.segment_sum(m, dst, num_segments=N_NODES)
    x_fin = _sp(_linear(h, p3['po'])) - _LOG2
    h = jax.nn.relu(_batchnorm(x_fin, params['bn_final'][0], params['bn_final'][1]))
    h = jnp.mean(h, axis=0, keepdims=True)
    out = _linear(h, params['fc'])
    return jnp.squeeze(out)


# speedup vs baseline: 1.0142x; 1.0142x over previous
import functools

import jax
import jax.numpy as jnp
from jax import lax
from jax.experimental import pallas as pl
from jax.experimental.pallas import tpu as pltpu
from jax.experimental.pallas import tpu_sc as plsc

N_NODES = 10000
N_EDGES = 160000
N_LG_EDGES = 640000
NODE_F = 64
EDGE_F = 40
ANGLE_F = 40
HIDDEN = 64

_LOG2 = 0.6931471805599453


def _sp(x):
    return jnp.logaddexp(x, 0.0)


def _edge_mlp_body(z_ref, w1_ref, b1_ref, w2_ref, b2_ref, o_ref, *, shifted):
    z = z_ref[...]
    h = jnp.dot(z, w1_ref[...], preferred_element_type=jnp.float32) + b1_ref[...]
    h = _sp(h)
    if shifted:
        h = h - _LOG2
    h = jnp.dot(h, w2_ref[...], preferred_element_type=jnp.float32) + b2_ref[...]
    if shifted:
        h = _sp(h) - _LOG2
    o_ref[...] = h


def _edge_mlp(z, p1, p2, shifted, block=8000):
    r, f = z.shape
    w1, b1 = p1
    w2, b2 = p2
    hid = w1.shape[0]
    out = w2.shape[0]
    return pl.pallas_call(
        functools.partial(_edge_mlp_body, shifted=shifted),
        grid=(r // block,),
        in_specs=[
            pl.BlockSpec((block, f), lambda i: (i, 0)),
            pl.BlockSpec((f, hid), lambda i: (0, 0)),
            pl.BlockSpec((1, hid), lambda i: (0, 0)),
            pl.BlockSpec((hid, out), lambda i: (0, 0)),
            pl.BlockSpec((1, out), lambda i: (0, 0)),
        ],
        out_specs=pl.BlockSpec((block, out), lambda i: (i, 0)),
        out_shape=jax.ShapeDtypeStruct((r, out), jnp.float32),
    )(z, w1.T, b1[None], w2.T, b2[None])


_K1_G = 128
_K1_NGRP = N_EDGES // _K1_G
_K1_PER_W = -(-_K1_NGRP // 32)
_K1_RPT = 624
_K1_RPT_LAST = N_NODES - 15 * _K1_RPT


def _k1_body(hv_hbm, he_hbm, src_hbm, dst_hbm, agg_hbm, m_hbm,
             sidx, didx, hevm, rows, zbuf, spmem, sem):
    c = lax.axis_index("c")
    s = lax.axis_index("s")
    w = s * 2 + c

    def zrow(i, carry):
        for k in range(4):
            zbuf[i, pl.ds(16 * k, 16)] = jnp.zeros((16,), jnp.float32)
        return carry

    lax.fori_loop(0, _K1_RPT_LAST, zrow, 0)

    @pl.when(s < 15)
    def _():
        pltpu.sync_copy(zbuf.at[pl.ds(0, _K1_RPT)],
                        spmem.at[pl.ds(s * _K1_RPT, _K1_RPT)])

    @pl.when(s == 15)
    def _():
        pltpu.sync_copy(zbuf.at[pl.ds(0, _K1_RPT_LAST)],
                        spmem.at[pl.ds(15 * _K1_RPT, _K1_RPT_LAST)])

    plsc.subcore_barrier()

    def grp(j, carry):
        g = w + 32 * j

        @pl.when(g < _K1_NGRP)
        def _():
            base = g * _K1_G
            pltpu.sync_copy(src_hbm.at[pl.ds(base, _K1_G)], sidx)
            pltpu.sync_copy(dst_hbm.at[pl.ds(base, _K1_G)], didx)
            pltpu.sync_copy(he_hbm.at[pl.ds(base, _K1_G)], hevm)
            pltpu.async_copy(hv_hbm.at[sidx], rows, sem).wait()

            def mrow(r, cc):
                for k in range(4):
                    sl = pl.ds(16 * k, 16)
                    rows[r, sl] = rows[r, sl] * hevm[r, sl]
                return cc

            lax.fori_loop(0, _K1_G, mrow, 0)
            pltpu.sync_copy(rows, m_hbm.at[pl.ds(base, _K1_G)])
            pltpu.sync_copy(rows, spmem.at[didx], add=True)

        return carry

    lax.fori_loop(0, _K1_PER_W, grp, 0)
    plsc.subcore_barrier()

    @pl.when(s < 15)
    def _():
        pltpu.sync_copy(
            spmem.at[pl.ds(s * _K1_RPT, _K1_RPT)],
            agg_hbm.at[pl.ds(c * N_NODES + s * _K1_RPT, _K1_RPT)])

    @pl.when(s == 15)
    def _():
        pltpu.sync_copy(
            spmem.at[pl.ds(15 * _K1_RPT, _K1_RPT_LAST)],
            agg_hbm.at[pl.ds(c * N_NODES + 15 * _K1_RPT, _K1_RPT_LAST)])


def _k1(hv, he, src, dst):
    f = pl.kernel(
        _k1_body,
        out_type=(
            jax.ShapeDtypeStruct((2 * N_NODES, NODE_F), jnp.float32),
            jax.ShapeDtypeStruct((N_EDGES, NODE_F), jnp.float32),
        ),
        mesh=plsc.VectorSubcoreMesh(core_axis_name="c", subcore_axis_name="s"),
        compiler_params=pltpu.CompilerParams(use_tc_tiling_on_sc=False),
        scratch_types=[
            pltpu.VMEM((_K1_G,), jnp.int32),
            pltpu.VMEM((_K1_G,), jnp.int32),
            pltpu.VMEM((_K1_G, NODE_F), jnp.float32),
            pltpu.VMEM((_K1_G, NODE_F), jnp.float32),
            pltpu.VMEM((_K1_RPT_LAST, NODE_F), jnp.float32),
            pltpu.VMEM_SHARED((N_NODES, NODE_F), jnp.float32),
            pltpu.SemaphoreType.DMA,
        ],
    )
    aggp, m = f(hv, he, src, dst)
    return aggp[:N_NODES] + aggp[N_NODES:], m


def _rbf(d, vmin, vmax, bins):
    centers = jnp.linspace(vmin, vmax, bins)
    gamma = 1.0 / ((vmax - vmin) / (bins - 1))
    return jnp.exp(-gamma * (d[:, None] - centers) ** 2)


def _linear(x, wb):
    w, b = wb
    return x @ w.T + b


def _batchnorm(x, gamma, beta, eps=1e-5):
    mu = x.mean(axis=0)
    var = x.var(axis=0)
    return gamma * (x - mu) / jnp.sqrt(var + eps) + beta


def kernel(atom_features, r, h_angle, edge_index, lg_edge_index, params):
    src, dst = edge_index[0], edge_index[1]
    lsrc, ldst = lg_edge_index[0], lg_edge_index[1]
    bondlength = jnp.sqrt(jnp.sum(r * r, axis=1))
    edge_feats = _rbf(bondlength, 0.0, 8.0, EDGE_F)
    x = _linear(atom_features, params['atom_emb'])
    x = jax.nn.relu(_batchnorm(x, params['bn'][0], params['bn'][1]))
    z = _rbf(h_angle, -1.0, 1.0, ANGLE_F)

    zp1 = _edge_mlp(z, params['conv1']['lg']['pe1'], params['conv1']['lg']['pe2'], True)
    zp2 = _edge_mlp(z, params['conv2']['lg']['pe1'], params['conv2']['lg']['pe2'], True)

    def clgn(p, x_in, y, zp):
        hv = _linear(x_in, p['pn'])
        he = _edge_mlp(y, p['pe1'], p['pe2'], False)
        agg, m = _k1(hv, he, src, dst)
        x_out = _sp(_linear(agg, p['po']))
        y_cat = jnp.concatenate([y, m], axis=1)
        lgp = p['lg']
        hv2 = _linear(y_cat, lgp['pn'])
        m2 = hv2[lsrc] * zp
        h2 = jax.ops.segment_sum(m2, ldst, num_segments=N_EDGES)
        y_out = _sp(_sp(_linear(h2, lgp['po'])) - _LOG2)
        return x_out, y_out

    x1, y1 = clgn(params['conv1'], x, edge_feats, zp1)
    x2, y2 = clgn(params['conv2'], x1, y1, zp2)

    p3 = params['conv3']
    hv = _linear(x2, p3['pn'])
    he = _edge_mlp(y2, p3['pe1'], p3['pe2'], True)
    h, _ = _k1(hv, he, src, dst)
    x_fin = _sp(_linear(h, p3['po'])) - _LOG2
    h = jax.nn.relu(_batchnorm(x_fin, params['bn_final'][0], params['bn_final'][1]))
    h = jnp.mean(h, axis=0, keepdims=True)
    out = _linear(h, params['fc'])
    return jnp.squeeze(out)
